# Initial kernel scaffold; baseline (speedup 1.0000x reference)
import functools

import jax
import jax.numpy as jnp
from jax import lax
from jax.experimental import pallas as pl
from jax.experimental.pallas import tpu as pltpu
from jax.experimental.pallas import tpu_sc as plsc

N = 10000
E = 320000
D = 128

NC = 2
NS = 16
NW = NC * NS
EPW = E // NW
C = 80
NCHUNK = EPW // C
ROWS_PER_TILE = N // NS
ZR = 125


def _spmm_body(h_hbm, row_hbm, col_hbm, ev_hbm, out_hbm,
               col_v, row_v, ev_v, rows_v, zero_v, accum_sh, sem):
    c = lax.axis_index("c")
    s = lax.axis_index("s")
    wid = c * NS + s
    base = wid * EPW

    @pl.loop(0, ZR)
    def _(r):
        for j in range(D // 16):
            zero_v[r, pl.ds(j * 16, 16)] = jnp.zeros((16,), jnp.float32)

    for k in range(ROWS_PER_TILE // ZR):
        pltpu.sync_copy(zero_v, accum_sh.at[pl.ds(s * ROWS_PER_TILE + k * ZR, ZR)])
    plsc.subcore_barrier()

    @pl.loop(0, NCHUNK)
    def _(i):
        off = base + i * C
        pltpu.sync_copy(col_hbm.at[pl.ds(off, C)], col_v)
        pltpu.sync_copy(ev_hbm.at[pl.ds(off, C)], ev_v)
        pltpu.sync_copy(row_hbm.at[pl.ds(off, C)], row_v)
        pltpu.async_copy(h_hbm.at[col_v], rows_v, sem).wait()

        @pl.loop(0, C)
        def _(e):
            idx = jnp.zeros((16,), jnp.int32) + e
            sv = plsc.load_gather(ev_v, [idx])
            for j in range(D // 16):
                sl = (e, pl.ds(j * 16, 16))
                rows_v[sl] = rows_v[sl] * sv

        pltpu.sync_copy(rows_v, accum_sh.at[row_v], add=True)

    plsc.subcore_barrier()

    r0 = s * ROWS_PER_TILE
    pltpu.sync_copy(accum_sh.at[pl.ds(r0, ROWS_PER_TILE)],
                    out_hbm.at[c].at[pl.ds(r0, ROWS_PER_TILE)])


def _spmm(h, row, col, ev):
    mesh = plsc.VectorSubcoreMesh(core_axis_name="c", subcore_axis_name="s")
    kern = pl.kernel(
        _spmm_body,
        out_type=jax.ShapeDtypeStruct((NC, N, D), jnp.float32),
        mesh=mesh,
        scratch_types=[
            pltpu.VMEM((C,), jnp.int32),
            pltpu.VMEM((C,), jnp.int32),
            pltpu.VMEM((C,), jnp.float32),
            pltpu.VMEM((C, D), jnp.float32),
            pltpu.VMEM((ZR, D), jnp.float32),
            pltpu.VMEM_SHARED((N, D), jnp.float32),
            pltpu.SemaphoreType.DMA,
        ],
    )
    return kern(h, row, col, ev)


def _dense1_body(p_ref, w1_ref, b1_ref, w2_ref, g_ref):
    t = p_ref[0] + p_ref[1]
    h1 = jnp.dot(t, w1_ref[...], preferred_element_type=jnp.float32,
                 precision=lax.Precision.HIGHEST)
    h1 = jnp.maximum(h1 + b1_ref[...], 0.0)
    g_ref[...] = jnp.dot(h1, w2_ref[...], preferred_element_type=jnp.float32,
                         precision=lax.Precision.HIGHEST)


def _dense2_body(q_ref, b2_ref, out_ref):
    out_ref[...] = q_ref[0] + q_ref[1] + b2_ref[...]


def kernel(x, edge_index, edge_values, W1, b1, W2, b2):
    row = edge_index[0]
    col = edge_index[1]

    p1 = _spmm(x, row, col, edge_values)

    BLK = 2000
    g = pl.pallas_call(
        _dense1_body,
        grid=(N // BLK,),
        in_specs=[
            pl.BlockSpec((NC, BLK, D), lambda i: (0, i, 0)),
            pl.BlockSpec((D, 256), lambda i: (0, 0)),
            pl.BlockSpec((1, 256), lambda i: (0, 0)),
            pl.BlockSpec((256, D), lambda i: (0, 0)),
        ],
        out_specs=pl.BlockSpec((BLK, D), lambda i: (i, 0)),
        out_shape=jax.ShapeDtypeStruct((N, D), jnp.float32),
    )(p1, W1, b1.reshape(1, 256), W2)

    p2 = _spmm(g, row, col, edge_values)

    out = pl.pallas_call(
        _dense2_body,
        grid=(N // BLK,),
        in_specs=[
            pl.BlockSpec((NC, BLK, D), lambda i: (0, i, 0)),
            pl.BlockSpec((1, D), lambda i: (0, 0)),
        ],
        out_specs=pl.BlockSpec((BLK, D), lambda i: (i, 0)),
        out_shape=jax.ShapeDtypeStruct((N, D), jnp.float32),
    )(p2, b2.reshape(1, D))

    return out


# speedup vs baseline: 4.1239x; 4.1239x over previous
import dataclasses
import functools

import jax
import jax.numpy as jnp
from jax import lax
from jax.experimental import pallas as pl
from jax.experimental.pallas import tpu as pltpu
from jax.experimental.pallas import tpu_sc as plsc

N = 10000
E = 320000
D = 128

NC = 2
NS = 16
NW = NC * NS
EPW = E // NW
C = 80
NCHUNK = EPW // C
RPT = 624
TAIL = N - NS * RPT
ZR = 208


def _spmm_body(h_hbm, row_hbm, col_hbm, ev_hbm, out_hbm,
               col_v, row_v, ev_v, rows_v, zero_v, accum_sh, sem):
    c = lax.axis_index("c")
    s = lax.axis_index("s")
    wid = c * NS + s
    base = wid * EPW

    @pl.loop(0, ZR)
    def _(r):
        for j in range(D // 16):
            zero_v[r, pl.ds(j * 16, 16)] = jnp.zeros((16,), jnp.float32)

    for k in range(RPT // ZR):
        pltpu.sync_copy(zero_v, accum_sh.at[pl.ds(s * RPT + k * ZR, ZR)])

    @pl.when(s == NS - 1)
    def _():
        pltpu.sync_copy(zero_v.at[pl.ds(0, TAIL)],
                        accum_sh.at[pl.ds(NS * RPT, TAIL)])

    plsc.subcore_barrier()

    @pl.loop(0, NCHUNK)
    def _(i):
        off = base + i * C
        pltpu.sync_copy(col_hbm.at[pl.ds(off, C)], col_v)
        pltpu.sync_copy(ev_hbm.at[pl.ds(off, C)], ev_v)
        pltpu.sync_copy(row_hbm.at[pl.ds(off, C)], row_v)
        pltpu.async_copy(h_hbm.at[col_v], rows_v, sem).wait()

        @pl.loop(0, C)
        def _(e):
            idx = jnp.zeros((16,), jnp.int32) + e
            sv = plsc.load_gather(ev_v, [idx])
            for j in range(D // 16):
                sl = (e, pl.ds(j * 16, 16))
                rows_v[sl] = rows_v[sl] * sv

        pltpu.sync_copy(rows_v, accum_sh.at[row_v], add=True)

    plsc.subcore_barrier()

    r0 = s * RPT
    pltpu.sync_copy(accum_sh.at[pl.ds(r0, RPT)],
                    out_hbm.at[c].at[pl.ds(r0, RPT)])

    @pl.when(s == NS - 1)
    def _():
        pltpu.sync_copy(accum_sh.at[pl.ds(NS * RPT, TAIL)],
                        out_hbm.at[c].at[pl.ds(NS * RPT, TAIL)])


_SC_PARAMS = pltpu.CompilerParams()
if "needs_layout_passes" in pltpu.CompilerParams.__dataclass_fields__:
    _SC_PARAMS = dataclasses.replace(_SC_PARAMS, needs_layout_passes=False)


def _spmm(h, row, col, ev):
    mesh = plsc.VectorSubcoreMesh(core_axis_name="c", subcore_axis_name="s")
    kern = pl.kernel(
        _spmm_body,
        out_type=jax.ShapeDtypeStruct((NC, N, D), jnp.float32),
        mesh=mesh,
        compiler_params=_SC_PARAMS,
        scratch_types=[
            pltpu.VMEM((C,), jnp.int32),
            pltpu.VMEM((C,), jnp.int32),
            pltpu.VMEM((C,), jnp.float32),
            pltpu.VMEM((C, D), jnp.float32),
            pltpu.VMEM((ZR, D), jnp.float32),
            pltpu.VMEM_SHARED((N, D), jnp.float32),
            pltpu.SemaphoreType.DMA,
        ],
    )
    return kern(h, row, col, ev)


def _dense1_body(p_ref, w1_ref, b1_ref, w2_ref, g_ref):
    t = p_ref[0] + p_ref[1]
    h1 = jnp.dot(t, w1_ref[...], preferred_element_type=jnp.float32,
                 precision=lax.Precision.HIGHEST)
    h1 = jnp.maximum(h1 + b1_ref[...], 0.0)
    g_ref[...] = jnp.dot(h1, w2_ref[...], preferred_element_type=jnp.float32,
                         precision=lax.Precision.HIGHEST)


def _dense2_body(q_ref, b2_ref, out_ref):
    out_ref[...] = q_ref[0] + q_ref[1] + b2_ref[...]


def kernel(x, edge_index, edge_values, W1, b1, W2, b2):
    row = edge_index[0]
    col = edge_index[1]

    p1 = _spmm(x, row, col, edge_values)

    BLK = 2000
    g = pl.pallas_call(
        _dense1_body,
        grid=(N // BLK,),
        in_specs=[
            pl.BlockSpec((NC, BLK, D), lambda i: (0, i, 0)),
            pl.BlockSpec((D, 256), lambda i: (0, 0)),
            pl.BlockSpec((1, 256), lambda i: (0, 0)),
            pl.BlockSpec((256, D), lambda i: (0, 0)),
        ],
        out_specs=pl.BlockSpec((BLK, D), lambda i: (i, 0)),
        out_shape=jax.ShapeDtypeStruct((N, D), jnp.float32),
    )(p1, W1, b1.reshape(1, 256), W2)

    p2 = _spmm(g, row, col, edge_values)

    out = pl.pallas_call(
        _dense2_body,
        grid=(N // BLK,),
        in_specs=[
            pl.BlockSpec((NC, BLK, D), lambda i: (0, i, 0)),
            pl.BlockSpec((1, D), lambda i: (0, 0)),
        ],
        out_specs=pl.BlockSpec((BLK, D), lambda i: (i, 0)),
        out_shape=jax.ShapeDtypeStruct((N, D), jnp.float32),
    )(p2, b2.reshape(1, D))

    return out


# speedup vs baseline: 10.6769x; 2.5891x over previous
import dataclasses
import functools

import jax
import jax.numpy as jnp
from jax import lax
from jax.experimental import pallas as pl
from jax.experimental.pallas import tpu as pltpu
from jax.experimental.pallas import tpu_sc as plsc

N = 10000
E = 320000
D = 128

NC = 2
NS = 16
NW = NC * NS
EPW = E // NW
C = 80
NCHUNK = EPW // C
RPT = 624
TAIL = N - NS * RPT
ZR = 48
SUPER = 25
NBLK = NCHUNK // SUPER


def _spmm_body(h_hbm, row_hbm, col_hbm, evf_hbm, out_hbm,
               col_b, row_b, ev_v0, ev_v1, rows0, rows1, zero_v, accum_sh,
               isem, gsem0, gsem1, ssem0, ssem1):
    c = lax.axis_index("c")
    s = lax.axis_index("s")
    wid = c * NS + s
    ebase = wid * EPW

    def i_start(b, slot):
        pltpu.async_copy(col_hbm.at[wid].at[b], col_b.at[slot], isem)
        pltpu.async_copy(row_hbm.at[wid].at[b], row_b.at[slot], isem)

    def i_wait(b, slot):
        pltpu.make_async_copy(col_hbm.at[wid].at[b], col_b.at[slot], isem).wait()
        pltpu.make_async_copy(row_hbm.at[wid].at[b], row_b.at[slot], isem).wait()

    i_start(0, 0)

    @pl.loop(0, ZR)
    def _(r):
        for j in range(D // 16):
            zero_v[r, pl.ds(j * 16, 16)] = jnp.zeros((16,), jnp.float32)

    for k in range(RPT // ZR):
        pltpu.sync_copy(zero_v, accum_sh.at[pl.ds(s * RPT + k * ZR, ZR)])

    @pl.when(s == NS - 1)
    def _():
        pltpu.sync_copy(zero_v.at[pl.ds(0, TAIL)],
                        accum_sh.at[pl.ds(NS * RPT, TAIL)])

    plsc.subcore_barrier()

    def g_start(sl, b, i, buf, evv, sem):
        pltpu.async_copy(h_hbm.at[col_b.at[sl].at[i]], buf, sem)
        pltpu.async_copy(evf_hbm.at[pl.ds(ebase + (b * SUPER + i) * C, C)],
                         evv, sem)

    def g_wait(sl, b, i, buf, evv, sem):
        pltpu.make_async_copy(h_hbm.at[col_b.at[sl].at[i]], buf, sem).wait()
        pltpu.make_async_copy(
            evf_hbm.at[pl.ds(ebase + (b * SUPER + i) * C, C)], evv, sem).wait()

    def s_start(sl, i, buf, sem):
        pltpu.async_copy(buf, accum_sh.at[row_b.at[sl].at[i]], sem, add=True)

    def s_wait(sl, i, buf, sem):
        pltpu.make_async_copy(buf, accum_sh.at[row_b.at[sl].at[i]], sem).wait()

    def scale(evv, buf):
        @pl.loop(0, C, step=2)
        def _(e):
            z = jnp.zeros((16,), jnp.int32)
            sv0 = plsc.load_gather(evv, [z + e])
            sv1 = plsc.load_gather(evv, [z + e + 1])
            for j in range(D // 16):
                sl0 = (e, pl.ds(j * 16, 16))
                sl1 = (e + 1, pl.ds(j * 16, 16))
                buf[sl0] = buf[sl0] * sv0
                buf[sl1] = buf[sl1] * sv1

    for b in range(NBLK):
        slot = b % 2
        i_wait(b, slot)
        if b + 1 < NBLK:
            i_start(b + 1, 1 - slot)

        g_start(slot, b, 0, rows0, ev_v0, gsem0)

        @pl.loop(0, (SUPER - 1) // 2)
        def _(k, slot=slot, b=b):
            i0 = 2 * k
            i1 = i0 + 1
            i2 = i0 + 2

            @pl.when(k > 0)
            def _():
                s_wait(slot, i1 - 2, rows1, ssem1)

            g_start(slot, b, i1, rows1, ev_v1, gsem1)
            g_wait(slot, b, i0, rows0, ev_v0, gsem0)
            scale(ev_v0, rows0)
            s_start(slot, i0, rows0, ssem0)

            s_wait(slot, i0, rows0, ssem0)
            g_start(slot, b, i2, rows0, ev_v0, gsem0)
            g_wait(slot, b, i1, rows1, ev_v1, gsem1)
            scale(ev_v1, rows1)
            s_start(slot, i1, rows1, ssem1)

        last = SUPER - 1
        g_wait(slot, b, last, rows0, ev_v0, gsem0)
        scale(ev_v0, rows0)
        s_start(slot, last, rows0, ssem0)
        s_wait(slot, last, rows0, ssem0)
        s_wait(slot, last - 1, rows1, ssem1)

    plsc.subcore_barrier()

    r0 = s * RPT
    pltpu.sync_copy(accum_sh.at[pl.ds(r0, RPT)],
                    out_hbm.at[c].at[pl.ds(r0, RPT)])

    @pl.when(s == NS - 1)
    def _():
        pltpu.sync_copy(accum_sh.at[pl.ds(NS * RPT, TAIL)],
                        out_hbm.at[c].at[pl.ds(NS * RPT, TAIL)])


_SC_PARAMS = pltpu.CompilerParams()
if "needs_layout_passes" in pltpu.CompilerParams.__dataclass_fields__:
    _SC_PARAMS = dataclasses.replace(_SC_PARAMS, needs_layout_passes=False)


def _spmm(h, row, col, ev):
    mesh = plsc.VectorSubcoreMesh(core_axis_name="c", subcore_axis_name="s")
    kern = pl.kernel(
        _spmm_body,
        out_type=jax.ShapeDtypeStruct((NC, N, D), jnp.float32),
        mesh=mesh,
        compiler_params=_SC_PARAMS,
        scratch_types=[
            pltpu.VMEM((2, SUPER, C), jnp.int32),
            pltpu.VMEM((2, SUPER, C), jnp.int32),
            pltpu.VMEM((C,), jnp.float32),
            pltpu.VMEM((C,), jnp.float32),
            pltpu.VMEM((C, D), jnp.float32),
            pltpu.VMEM((C, D), jnp.float32),
            pltpu.VMEM((ZR, D), jnp.float32),
            pltpu.VMEM_SHARED((N, D), jnp.float32),
            pltpu.SemaphoreType.DMA,
            pltpu.SemaphoreType.DMA,
            pltpu.SemaphoreType.DMA,
            pltpu.SemaphoreType.DMA,
            pltpu.SemaphoreType.DMA,
        ],
    )
    return kern(h, row.reshape(NW, NBLK, SUPER, C),
                col.reshape(NW, NBLK, SUPER, C), ev)


def _dense1_body(p_ref, w1_ref, b1_ref, w2_ref, g_ref):
    t = p_ref[0] + p_ref[1]
    h1 = jnp.dot(t, w1_ref[...], preferred_element_type=jnp.float32,
                 precision=lax.Precision.HIGHEST)
    h1 = jnp.maximum(h1 + b1_ref[...], 0.0)
    g_ref[...] = jnp.dot(h1, w2_ref[...], preferred_element_type=jnp.float32,
                         precision=lax.Precision.HIGHEST)


def _dense2_body(q_ref, b2_ref, out_ref):
    out_ref[...] = q_ref[0] + q_ref[1] + b2_ref[...]


def kernel(x, edge_index, edge_values, W1, b1, W2, b2):
    row = edge_index[0]
    col = edge_index[1]

    p1 = _spmm(x, row, col, edge_values)

    BLK = 2000
    g = pl.pallas_call(
        _dense1_body,
        grid=(N // BLK,),
        in_specs=[
            pl.BlockSpec((NC, BLK, D), lambda i: (0, i, 0)),
            pl.BlockSpec((D, 256), lambda i: (0, 0)),
            pl.BlockSpec((1, 256), lambda i: (0, 0)),
            pl.BlockSpec((256, D), lambda i: (0, 0)),
        ],
        out_specs=pl.BlockSpec((BLK, D), lambda i: (i, 0)),
        out_shape=jax.ShapeDtypeStruct((N, D), jnp.float32),
    )(p1, W1, b1.reshape(1, 256), W2)

    p2 = _spmm(g, row, col, edge_values)

    out = pl.pallas_call(
        _dense2_body,
        grid=(N // BLK,),
        in_specs=[
            pl.BlockSpec((NC, BLK, D), lambda i: (0, i, 0)),
            pl.BlockSpec((1, D), lambda i: (0, 0)),
        ],
        out_specs=pl.BlockSpec((BLK, D), lambda i: (i, 0)),
        out_shape=jax.ShapeDtypeStruct((N, D), jnp.float32),
    )(p2, b2.reshape(1, D))

    return out


# speedup vs baseline: 11.7612x; 1.1016x over previous
import dataclasses
import functools

import jax
import jax.numpy as jnp
from jax import lax
from jax.experimental import pallas as pl
from jax.experimental.pallas import tpu as pltpu
from jax.experimental.pallas import tpu_sc as plsc

N = 10000
E = 320000
D = 128

NC = 2
NS = 16
NW = NC * NS
EPW = E // NW
C = 80
NCHUNK = EPW // C
RPT = 624
TAIL = N - NS * RPT
ZR = 16
SUPER = 25
NBLK = NCHUNK // SUPER


def _spmm_body(h_hbm, row_hbm, col_hbm, evf_hbm, out_hbm,
               col_b, row_b, ev_v0, ev_v1, ev_v2, rows0, rows1, rows2,
               zero_v, accum_sh,
               isem, zsem, gsem0, gsem1, gsem2, ssem0, ssem1, ssem2):
    c = lax.axis_index("c")
    s = lax.axis_index("s")
    wid = c * NS + s
    ebase = wid * EPW

    def i_start(b, slot):
        pltpu.async_copy(col_hbm.at[wid].at[b], col_b.at[slot], isem)
        pltpu.async_copy(row_hbm.at[wid].at[b], row_b.at[slot], isem)

    def i_wait(b, slot):
        pltpu.make_async_copy(col_hbm.at[wid].at[b], col_b.at[slot], isem).wait()
        pltpu.make_async_copy(row_hbm.at[wid].at[b], row_b.at[slot], isem).wait()

    i_start(0, 0)

    @pl.loop(0, ZR)
    def _(r):
        for j in range(D // 16):
            zero_v[r, pl.ds(j * 16, 16)] = jnp.zeros((16,), jnp.float32)

    @pl.loop(0, RPT // ZR)
    def _(k):
        pltpu.async_copy(zero_v, accum_sh.at[pl.ds(s * RPT + k * ZR, ZR)], zsem)

    @pl.when(s == NS - 1)
    def _():
        pltpu.sync_copy(zero_v.at[pl.ds(0, TAIL)],
                        accum_sh.at[pl.ds(NS * RPT, TAIL)])

    @pl.loop(0, RPT // ZR)
    def _(k):
        pltpu.make_async_copy(
            zero_v, accum_sh.at[pl.ds(s * RPT + k * ZR, ZR)], zsem).wait()

    plsc.subcore_barrier()

    def g_start(sl, b, i, buf, evv, sem):
        pltpu.async_copy(h_hbm.at[col_b.at[sl].at[i]], buf, sem)
        pltpu.async_copy(evf_hbm.at[pl.ds(ebase + (b * SUPER + i) * C, C)],
                         evv, sem)

    def g_wait(sl, b, i, buf, evv, sem):
        pltpu.make_async_copy(h_hbm.at[col_b.at[sl].at[i]], buf, sem).wait()
        pltpu.make_async_copy(
            evf_hbm.at[pl.ds(ebase + (b * SUPER + i) * C, C)], evv, sem).wait()

    def s_start(sl, i, buf, sem):
        pltpu.async_copy(buf, accum_sh.at[row_b.at[sl].at[i]], sem, add=True)

    def s_wait(sl, i, buf, sem):
        pltpu.make_async_copy(buf, accum_sh.at[row_b.at[sl].at[i]], sem).wait()

    def scale(evv, buf):
        @pl.loop(0, C, step=2)
        def _(e):
            z = jnp.zeros((16,), jnp.int32)
            sv0 = plsc.load_gather(evv, [z + e])
            sv1 = plsc.load_gather(evv, [z + e + 1])
            for j in range(D // 16):
                sl0 = (e, pl.ds(j * 16, 16))
                sl1 = (e + 1, pl.ds(j * 16, 16))
                buf[sl0] = buf[sl0] * sv0
                buf[sl1] = buf[sl1] * sv1

    bufs = (rows0, rows1, rows2)
    evs = (ev_v0, ev_v1, ev_v2)
    gsems = (gsem0, gsem1, gsem2)
    ssems = (ssem0, ssem1, ssem2)

    def step(slot, b, ci, p0, p2, guard_first):
        g_wait(slot, b, ci, bufs[p0], evs[p0], gsems[p0])
        scale(evs[p0], bufs[p0])
        s_start(slot, ci, bufs[p0], ssems[p0])
        if guard_first:
            @pl.when(ci >= 1)
            def _():
                s_wait(slot, ci - 1, bufs[p2], ssems[p2])
        else:
            s_wait(slot, ci - 1, bufs[p2], ssems[p2])
        g_start(slot, b, ci + 2, bufs[p2], evs[p2], gsems[p2])

    for b in range(NBLK):
        slot = b % 2
        i_wait(b, slot)
        if b + 1 < NBLK:
            i_start(b + 1, 1 - slot)

        g_start(slot, b, 0, rows0, ev_v0, gsem0)
        g_start(slot, b, 1, rows1, ev_v1, gsem1)

        @pl.loop(0, (SUPER - 4) // 3)
        def _(k, slot=slot, b=b):
            c0 = 3 * k
            step(slot, b, c0, 0, 2, True)
            step(slot, b, c0 + 1, 1, 0, False)
            step(slot, b, c0 + 2, 2, 1, False)

        g_wait(slot, b, 21, rows0, ev_v0, gsem0)
        scale(ev_v0, rows0)
        s_start(slot, 21, rows0, ssem0)
        s_wait(slot, 20, rows2, ssem2)
        g_start(slot, b, 23, rows2, ev_v2, gsem2)

        g_wait(slot, b, 22, rows1, ev_v1, gsem1)
        scale(ev_v1, rows1)
        s_start(slot, 22, rows1, ssem1)
        s_wait(slot, 21, rows0, ssem0)
        g_start(slot, b, 24, rows0, ev_v0, gsem0)

        g_wait(slot, b, 23, rows2, ev_v2, gsem2)
        scale(ev_v2, rows2)
        s_start(slot, 23, rows2, ssem2)

        g_wait(slot, b, 24, rows0, ev_v0, gsem0)
        scale(ev_v0, rows0)
        s_start(slot, 24, rows0, ssem0)

        s_wait(slot, 22, rows1, ssem1)
        s_wait(slot, 23, rows2, ssem2)
        s_wait(slot, 24, rows0, ssem0)

    plsc.subcore_barrier()

    r0 = s * RPT
    pltpu.sync_copy(accum_sh.at[pl.ds(r0, RPT)],
                    out_hbm.at[c].at[pl.ds(r0, RPT)])

    @pl.when(s == NS - 1)
    def _():
        pltpu.sync_copy(accum_sh.at[pl.ds(NS * RPT, TAIL)],
                        out_hbm.at[c].at[pl.ds(NS * RPT, TAIL)])


_SC_PARAMS = pltpu.CompilerParams()
if "needs_layout_passes" in pltpu.CompilerParams.__dataclass_fields__:
    _SC_PARAMS = dataclasses.replace(_SC_PARAMS, needs_layout_passes=False)


def _spmm(h, row, col, ev):
    mesh = plsc.VectorSubcoreMesh(core_axis_name="c", subcore_axis_name="s")
    kern = pl.kernel(
        _spmm_body,
        out_type=jax.ShapeDtypeStruct((NC, N, D), jnp.float32),
        mesh=mesh,
        compiler_params=_SC_PARAMS,
        scratch_types=[
            pltpu.VMEM((2, SUPER, C), jnp.int32),
            pltpu.VMEM((2, SUPER, C), jnp.int32),
            pltpu.VMEM((C,), jnp.float32),
            pltpu.VMEM((C,), jnp.float32),
            pltpu.VMEM((C,), jnp.float32),
            pltpu.VMEM((C, D), jnp.float32),
            pltpu.VMEM((C, D), jnp.float32),
            pltpu.VMEM((C, D), jnp.float32),
            pltpu.VMEM((ZR, D), jnp.float32),
            pltpu.VMEM_SHARED((N, D), jnp.float32),
            pltpu.SemaphoreType.DMA,
            pltpu.SemaphoreType.DMA,
            pltpu.SemaphoreType.DMA,
            pltpu.SemaphoreType.DMA,
            pltpu.SemaphoreType.DMA,
            pltpu.SemaphoreType.DMA,
            pltpu.SemaphoreType.DMA,
            pltpu.SemaphoreType.DMA,
        ],
    )
    return kern(h, row.reshape(NW, NBLK, SUPER, C),
                col.reshape(NW, NBLK, SUPER, C), ev)


def _dense1_body(p_ref, w1_ref, b1_ref, w2_ref, g_ref):
    t = p_ref[0] + p_ref[1]
    h1 = jnp.dot(t, w1_ref[...], preferred_element_type=jnp.float32,
                 precision=lax.Precision.HIGHEST)
    h1 = jnp.maximum(h1 + b1_ref[...], 0.0)
    g_ref[...] = jnp.dot(h1, w2_ref[...], preferred_element_type=jnp.float32,
                         precision=lax.Precision.HIGHEST)


def _dense2_body(q_ref, b2_ref, out_ref):
    out_ref[...] = q_ref[0] + q_ref[1] + b2_ref[...]


def kernel(x, edge_index, edge_values, W1, b1, W2, b2):
    row = edge_index[0]
    col = edge_index[1]

    p1 = _spmm(x, row, col, edge_values)

    BLK = 2000
    g = pl.pallas_call(
        _dense1_body,
        grid=(N // BLK,),
        in_specs=[
            pl.BlockSpec((NC, BLK, D), lambda i: (0, i, 0)),
            pl.BlockSpec((D, 256), lambda i: (0, 0)),
            pl.BlockSpec((1, 256), lambda i: (0, 0)),
            pl.BlockSpec((256, D), lambda i: (0, 0)),
        ],
        out_specs=pl.BlockSpec((BLK, D), lambda i: (i, 0)),
        out_shape=jax.ShapeDtypeStruct((N, D), jnp.float32),
    )(p1, W1, b1.reshape(1, 256), W2)

    p2 = _spmm(g, row, col, edge_values)

    out = pl.pallas_call(
        _dense2_body,
        grid=(N // BLK,),
        in_specs=[
            pl.BlockSpec((NC, BLK, D), lambda i: (0, i, 0)),
            pl.BlockSpec((1, D), lambda i: (0, 0)),
        ],
        out_specs=pl.BlockSpec((BLK, D), lambda i: (i, 0)),
        out_shape=jax.ShapeDtypeStruct((N, D), jnp.float32),
    )(p2, b2.reshape(1, D))

    return out


# speedup vs baseline: 12.1861x; 1.0361x over previous
import dataclasses
import functools

import jax
import jax.numpy as jnp
from jax import lax
from jax.experimental import pallas as pl
from jax.experimental.pallas import tpu as pltpu
from jax.experimental.pallas import tpu_sc as plsc

N = 10000
E = 320000
D = 128

NC = 2
NS = 16
NW = NC * NS
EPW = E // NW
C = 80
NCHUNK = EPW // C
RPT = 624
TAIL = N - NS * RPT
ZR = 16
SUPER = 25
NBLK = NCHUNK // SUPER


def _spmm_body(h_hbm, row_hbm, col_hbm, evf_hbm, out_hbm,
               col_b, row_b, ev_v0, ev_v1, ev_v2, rows0, rows1, rows2,
               zero_v, accum_sh,
               isem, zsem, gsem0, gsem1, gsem2, ssem0, ssem1, ssem2):
    c = lax.axis_index("c")
    s = lax.axis_index("s")
    wid = c * NS + s
    ebase = wid * EPW

    def i_start(b, slot):
        pltpu.async_copy(col_hbm.at[wid].at[b], col_b.at[slot], isem)
        pltpu.async_copy(row_hbm.at[wid].at[b], row_b.at[slot], isem)

    def i_wait(b, slot):
        pltpu.make_async_copy(col_hbm.at[wid].at[b], col_b.at[slot], isem).wait()
        pltpu.make_async_copy(row_hbm.at[wid].at[b], row_b.at[slot], isem).wait()

    i_start(0, 0)

    @pl.loop(0, ZR)
    def _(r):
        for j in range(D // 16):
            zero_v[r, pl.ds(j * 16, 16)] = jnp.zeros((16,), jnp.float32)

    @pl.loop(0, RPT // ZR)
    def _(k):
        pltpu.async_copy(zero_v, accum_sh.at[pl.ds(s * RPT + k * ZR, ZR)], zsem)

    @pl.when(s == NS - 1)
    def _():
        pltpu.sync_copy(zero_v.at[pl.ds(0, TAIL)],
                        accum_sh.at[pl.ds(NS * RPT, TAIL)])

    @pl.loop(0, RPT // ZR)
    def _(k):
        pltpu.make_async_copy(
            zero_v, accum_sh.at[pl.ds(s * RPT + k * ZR, ZR)], zsem).wait()

    plsc.subcore_barrier()

    def g_start(sl, b, i, buf, evv, sem):
        pltpu.async_copy(h_hbm.at[col_b.at[sl].at[i]], buf, sem)
        pltpu.async_copy(evf_hbm.at[pl.ds(ebase + (b * SUPER + i) * C, C)],
                         evv, sem)

    def g_wait(sl, b, i, buf, evv, sem):
        pltpu.make_async_copy(h_hbm.at[col_b.at[sl].at[i]], buf, sem).wait()
        pltpu.make_async_copy(
            evf_hbm.at[pl.ds(ebase + (b * SUPER + i) * C, C)], evv, sem).wait()

    def s_start(sl, i, buf, sem):
        pltpu.async_copy(buf, accum_sh.at[row_b.at[sl].at[i]], sem, add=True)

    def s_wait(sl, i, buf, sem):
        pltpu.make_async_copy(buf, accum_sh.at[row_b.at[sl].at[i]], sem).wait()

    def scale(evv, buf):
        @plsc.parallel_loop(0, C, step=2, unroll=2)
        def _(e):
            z = jnp.zeros((16,), jnp.int32)
            sv0 = plsc.load_gather(evv, [z + e])
            sv1 = plsc.load_gather(evv, [z + e + 1])
            for j in range(D // 16):
                sl0 = (e, pl.ds(j * 16, 16))
                sl1 = (e + 1, pl.ds(j * 16, 16))
                buf[sl0] = buf[sl0] * sv0
                buf[sl1] = buf[sl1] * sv1

    bufs = (rows0, rows1, rows2)
    evs = (ev_v0, ev_v1, ev_v2)
    gsems = (gsem0, gsem1, gsem2)
    ssems = (ssem0, ssem1, ssem2)

    def step(slot, b, ci, p0, p2, guard_first):
        g_wait(slot, b, ci, bufs[p0], evs[p0], gsems[p0])
        scale(evs[p0], bufs[p0])
        s_start(slot, ci, bufs[p0], ssems[p0])
        if guard_first:
            @pl.when(ci >= 1)
            def _():
                s_wait(slot, ci - 1, bufs[p2], ssems[p2])
        else:
            s_wait(slot, ci - 1, bufs[p2], ssems[p2])
        g_start(slot, b, ci + 2, bufs[p2], evs[p2], gsems[p2])

    for b in range(NBLK):
        slot = b % 2
        i_wait(b, slot)
        if b + 1 < NBLK:
            i_start(b + 1, 1 - slot)

        g_start(slot, b, 0, rows0, ev_v0, gsem0)
        g_start(slot, b, 1, rows1, ev_v1, gsem1)

        @pl.loop(0, (SUPER - 4) // 3)
        def _(k, slot=slot, b=b):
            c0 = 3 * k
            step(slot, b, c0, 0, 2, True)
            step(slot, b, c0 + 1, 1, 0, False)
            step(slot, b, c0 + 2, 2, 1, False)

        g_wait(slot, b, 21, rows0, ev_v0, gsem0)
        scale(ev_v0, rows0)
        s_start(slot, 21, rows0, ssem0)
        s_wait(slot, 20, rows2, ssem2)
        g_start(slot, b, 23, rows2, ev_v2, gsem2)

        g_wait(slot, b, 22, rows1, ev_v1, gsem1)
        scale(ev_v1, rows1)
        s_start(slot, 22, rows1, ssem1)
        s_wait(slot, 21, rows0, ssem0)
        g_start(slot, b, 24, rows0, ev_v0, gsem0)

        g_wait(slot, b, 23, rows2, ev_v2, gsem2)
        scale(ev_v2, rows2)
        s_start(slot, 23, rows2, ssem2)

        g_wait(slot, b, 24, rows0, ev_v0, gsem0)
        scale(ev_v0, rows0)
        s_start(slot, 24, rows0, ssem0)

        s_wait(slot, 22, rows1, ssem1)
        s_wait(slot, 23, rows2, ssem2)
        s_wait(slot, 24, rows0, ssem0)

    plsc.subcore_barrier()

    r0 = s * RPT
    pltpu.sync_copy(accum_sh.at[pl.ds(r0, RPT)],
                    out_hbm.at[c].at[pl.ds(r0, RPT)])

    @pl.when(s == NS - 1)
    def _():
        pltpu.sync_copy(accum_sh.at[pl.ds(NS * RPT, TAIL)],
                        out_hbm.at[c].at[pl.ds(NS * RPT, TAIL)])


_SC_PARAMS = pltpu.CompilerParams()
if "needs_layout_passes" in pltpu.CompilerParams.__dataclass_fields__:
    _SC_PARAMS = dataclasses.replace(_SC_PARAMS, needs_layout_passes=False)


def _spmm(h, row, col, ev):
    mesh = plsc.VectorSubcoreMesh(core_axis_name="c", subcore_axis_name="s")
    kern = pl.kernel(
        _spmm_body,
        out_type=jax.ShapeDtypeStruct((NC, N, D), jnp.float32),
        mesh=mesh,
        compiler_params=_SC_PARAMS,
        scratch_types=[
            pltpu.VMEM((2, SUPER, C), jnp.int32),
            pltpu.VMEM((2, SUPER, C), jnp.int32),
            pltpu.VMEM((C,), jnp.float32),
            pltpu.VMEM((C,), jnp.float32),
            pltpu.VMEM((C,), jnp.float32),
            pltpu.VMEM((C, D), jnp.float32),
            pltpu.VMEM((C, D), jnp.float32),
            pltpu.VMEM((C, D), jnp.float32),
            pltpu.VMEM((ZR, D), jnp.float32),
            pltpu.VMEM_SHARED((N, D), jnp.float32),
            pltpu.SemaphoreType.DMA,
            pltpu.SemaphoreType.DMA,
            pltpu.SemaphoreType.DMA,
            pltpu.SemaphoreType.DMA,
            pltpu.SemaphoreType.DMA,
            pltpu.SemaphoreType.DMA,
            pltpu.SemaphoreType.DMA,
            pltpu.SemaphoreType.DMA,
        ],
    )
    return kern(h, row.reshape(NW, NBLK, SUPER, C),
                col.reshape(NW, NBLK, SUPER, C), ev)


def _dense1_body(p_ref, w1_ref, b1_ref, w2_ref, g_ref):
    t = p_ref[0] + p_ref[1]
    h1 = jnp.dot(t, w1_ref[...], preferred_element_type=jnp.float32,
                 precision=lax.Precision.HIGHEST)
    h1 = jnp.maximum(h1 + b1_ref[...], 0.0)
    g_ref[...] = jnp.dot(h1, w2_ref[...], preferred_element_type=jnp.float32,
                         precision=lax.Precision.HIGHEST)


def _dense2_body(q_ref, b2_ref, out_ref):
    out_ref[...] = q_ref[0] + q_ref[1] + b2_ref[...]


def kernel(x, edge_index, edge_values, W1, b1, W2, b2):
    row = edge_index[0]
    col = edge_index[1]

    p1 = _spmm(x, row, col, edge_values)

    BLK = 2000
    g = pl.pallas_call(
        _dense1_body,
        grid=(N // BLK,),
        in_specs=[
            pl.BlockSpec((NC, BLK, D), lambda i: (0, i, 0)),
            pl.BlockSpec((D, 256), lambda i: (0, 0)),
            pl.BlockSpec((1, 256), lambda i: (0, 0)),
            pl.BlockSpec((256, D), lambda i: (0, 0)),
        ],
        out_specs=pl.BlockSpec((BLK, D), lambda i: (i, 0)),
        out_shape=jax.ShapeDtypeStruct((N, D), jnp.float32),
    )(p1, W1, b1.reshape(1, 256), W2)

    p2 = _spmm(g, row, col, edge_values)

    out = pl.pallas_call(
        _dense2_body,
        grid=(N // BLK,),
        in_specs=[
            pl.BlockSpec((NC, BLK, D), lambda i: (0, i, 0)),
            pl.BlockSpec((1, D), lambda i: (0, 0)),
        ],
        out_specs=pl.BlockSpec((BLK, D), lambda i: (i, 0)),
        out_shape=jax.ShapeDtypeStruct((N, D), jnp.float32),
    )(p2, b2.reshape(1, D))

    return out
